# baseline (speedup 1.0000x reference)
import functools

import jax
import jax.numpy as jnp
from jax import lax
from jax.experimental import pallas as pl
from jax.experimental.pallas import tpu as pltpu
from jax.experimental.pallas import tpu_sc as plsc

N_NODES = 50000
N_EDGES = 800000
NUM_TOKENS = 157
FW = 16
NC, NS = 2, 16
NW = NC * NS
CH = 128
ROWS = N_EDGES // CH
ROWS_PW = ROWS // NW
ROWS_REM = ROWS - ROWS_PW * NW
NB = 16
NRND = 12
NBD = 15
NP = 50048
NPC = NP // NS
NPK = NP // 8
BN = 2048
NG = 25


def _sc_mesh():
    return plsc.VectorSubcoreMesh(core_axis_name="c", subcore_axis_name="s")


def _sc_params():
    return pltpu.CompilerParams(use_tc_tiling_on_sc=False)


def _zero_shared(zbuf, shared, s):
    @pl.loop(0, 128)
    def _(i):
        zbuf[i, :] = jnp.zeros((FW,), jnp.float32)

    @pl.loop(0, 24)
    def _(k):
        pltpu.sync_copy(zbuf, shared.at[pl.ds(s * NPC + k * 128, 128)])

    pltpu.sync_copy(zbuf.at[pl.ds(0, 104)],
                    shared.at[pl.ds(s * NPC + 24 * 128, 104)])


def _make_sc_deg():
    @functools.partial(
        pl.kernel,
        out_type=jax.ShapeDtypeStruct((NC, NP, FW), jnp.float32),
        mesh=_sc_mesh(),
        compiler_params=_sc_params(),
        scratch_types=[
            pltpu.VMEM((ROWS_PW, CH), jnp.int32),
            pltpu.VMEM((CH, FW), jnp.float32),
            pltpu.VMEM((128, FW), jnp.float32),
            pltpu.VMEM((CH,), jnp.int32),
            pltpu.VMEM_SHARED((NP, FW), jnp.float32),
            pltpu.SemaphoreType.DMA,
        ],
    )
    def sc_deg(eidx_hbm, deg_hbm, sidx, ones, zbuf, xsrc, shared, ssem):
        c = lax.axis_index("c")
        s = lax.axis_index("s")
        w = c * NS + s

        _zero_shared(zbuf, shared, s)

        @pl.loop(0, CH)
        def _(i):
            ones[i, :] = jnp.ones((FW,), jnp.float32)

        plsc.subcore_barrier()

        pltpu.sync_copy(eidx_hbm.at[pl.ds(w * ROWS_PW, ROWS_PW)], sidx)

        @pl.loop(0, ROWS_PW // NBD)
        def _(t):
            hs = [pltpu.async_copy(ones, shared.at[sidx.at[t * NBD + b]],
                                   ssem, add=True)
                  for b in range(NBD)]
            for h in hs:
                h.wait()

        @pl.when(w < ROWS_REM)
        def _():
            pltpu.sync_copy(eidx_hbm.at[NW * ROWS_PW + w], xsrc)
            pltpu.sync_copy(ones, shared.at[xsrc], add=True)

        plsc.subcore_barrier()
        pltpu.sync_copy(shared.at[pl.ds(s * NPC, NPC)],
                        deg_hbm.at[c, pl.ds(s * NPC, NPC)])

    return sc_deg


def _make_sc_edge():
    @functools.partial(
        pl.kernel,
        out_type=jax.ShapeDtypeStruct((NC, NP, FW), jnp.float32),
        mesh=_sc_mesh(),
        compiler_params=_sc_params(),
        scratch_types=[
            pltpu.VMEM((2, NB, CH), jnp.int32),
            pltpu.VMEM((2, NB, CH), jnp.int32),
            pltpu.VMEM((NB, CH, FW), jnp.float32),
            pltpu.VMEM((128, FW), jnp.float32),
            pltpu.VMEM((CH,), jnp.int32),
            pltpu.VMEM((CH,), jnp.int32),
            pltpu.VMEM_SHARED((NP, FW), jnp.float32),
            pltpu.SemaphoreType.DMA,
            pltpu.SemaphoreType.DMA,
            pltpu.SemaphoreType.DMA,
        ],
    )
    def sc_edge(eidx_hbm, bd_hbm, s_hbm,
                sidx, didx, rows, zbuf, xsrc, xdst, shared,
                gsem, ssem, isem):
        c = lax.axis_index("c")
        s = lax.axis_index("s")
        w = c * NS + s
        base = w * ROWS_PW

        def load_idx(t, slot):
            pltpu.async_copy(eidx_hbm.at[pl.ds(base + t * NB, NB)],
                             sidx.at[slot], isem)
            pltpu.async_copy(eidx_hbm.at[pl.ds(ROWS + base + t * NB, NB)],
                             didx.at[slot], isem)

        def wait_idx(slot):
            pltpu.make_async_copy(eidx_hbm.at[pl.ds(0, NB)],
                                  sidx.at[slot], isem).wait()
            pltpu.make_async_copy(eidx_hbm.at[pl.ds(0, NB)],
                                  didx.at[slot], isem).wait()

        load_idx(0, 0)
        _zero_shared(zbuf, shared, s)
        plsc.subcore_barrier()

        @pl.loop(0, NRND // 2)
        def _(tt):
            for slot in (0, 1):
                t = tt * 2 + slot
                wait_idx(slot)

                @pl.when(t < NRND - 1)
                def _():
                    load_idx(t + 1, 1 - slot)

                ghs = [pltpu.async_copy(bd_hbm.at[sidx.at[slot, b]],
                                        rows.at[b], gsem)
                       for b in range(NB)]
                shs = []
                for b in range(NB):
                    ghs[b].wait()
                    shs.append(pltpu.async_copy(
                        rows.at[b], shared.at[didx.at[slot, b]],
                        ssem, add=True))
                for h in shs:
                    h.wait()

        ntail = ROWS_PW - NRND * NB
        pltpu.sync_copy(eidx_hbm.at[pl.ds(base + NRND * NB, ntail)],
                        sidx.at[0, pl.ds(0, ntail)])
        pltpu.sync_copy(eidx_hbm.at[pl.ds(ROWS + base + NRND * NB, ntail)],
                        didx.at[0, pl.ds(0, ntail)])
        for b in range(ntail):
            pltpu.sync_copy(bd_hbm.at[sidx.at[0, b]], rows.at[b])
            pltpu.sync_copy(rows.at[b], shared.at[didx.at[0, b]],
                            add=True)

        @pl.when(w < ROWS_REM)
        def _():
            pltpu.sync_copy(eidx_hbm.at[NW * ROWS_PW + w], xsrc)
            pltpu.sync_copy(eidx_hbm.at[ROWS + NW * ROWS_PW + w], xdst)
            pltpu.sync_copy(bd_hbm.at[xsrc], rows.at[0])
            pltpu.sync_copy(rows.at[0], shared.at[xdst], add=True)

        plsc.subcore_barrier()
        pltpu.sync_copy(shared.at[pl.ds(s * NPC, NPC)],
                        s_hbm.at[c, pl.ds(s * NPC, NPC)])

    return sc_edge


def _tc_encbd_body(xt_ref, deg_ref, a_ref, b_ref, o_ref):
    a = lax.dot_general(
        xt_ref[...], a_ref[...], (((0,), (0,)), ((), ())),
        preferred_element_type=jnp.float32) + b_ref[...]
    d = deg_ref[0, :, 0:1] + deg_ref[1, :, 0:1]
    dis = jnp.where(d > 0.0, 1.0 / jnp.sqrt(jnp.maximum(d, 1.0)), 0.0)
    col = lax.broadcasted_iota(jnp.int32, (BN, FW), 1)
    o_ref[...] = jnp.where(col == 10, dis, a * dis)


def _tc_final_body(xt_ref, bd_ref, s_ref, u_ref, w1_ref, bias_ref,
                   lv_ref, lb_ref, o_ref):
    dis = bd_ref[:, 10:11]
    sw = jnp.dot(s_ref[0] + s_ref[1], w1_ref[...],
                 preferred_element_type=jnp.float32)
    g = (lax.dot_general(
            xt_ref[...], u_ref[...], (((0,), (0,)), ((), ())),
            preferred_element_type=jnp.float32)
         - dis * sw + bias_ref[...])
    z = jax.nn.sigmoid(g[:, 0:64])
    ht = jnp.tanh(g[:, 64:128])
    h = jax.nn.relu((1.0 - z) * ht)
    r = jnp.sum(h * lv_ref[...], axis=1, keepdims=True) + lb_ref[...]
    o_ref[...] = jax.nn.sigmoid(r)


def kernel(x, edge_index, enc_W, enc_b, Wxz0, Wxz1, bxz, Whz0, Whz1, bhz,
           Wxr0, Wxr1, bxr, Whr0, Whr1, bhr, Wxh0, Wxh1, bxh,
           Whh0, Whh1, bhh, lin_W, lin_b):
    f32 = jnp.float32
    A = jnp.zeros((2 * NUM_TOKENS + 2, FW), f32)
    A = A.at[0:NUM_TOKENS, 0:4].set(enc_W)
    A = A.at[NUM_TOKENS, 4].set(1.0)
    A = A.at[NUM_TOKENS + 1:2 * NUM_TOKENS + 1, 5:9].set(enc_W)
    A = A.at[2 * NUM_TOKENS + 1, 9].set(1.0)
    b_a = jnp.zeros((1, FW), f32)
    b_a = b_a.at[0, 0:4].set(enc_b).at[0, 5:9].set(enc_b)

    W0 = jnp.zeros((FW, 128), f32)
    W0 = W0.at[0:10, 0:64].set(Wxz0).at[0:10, 64:128].set(Wxh0)
    W1 = jnp.zeros((FW, 128), f32)
    W1 = W1.at[0:10, 0:64].set(Wxz1).at[0:10, 64:128].set(Wxh1)
    bias = jnp.concatenate([bxz + bhz, bxh + bhh]).reshape(1, 128)
    lv = lin_W.reshape(1, 64)
    lb = lin_b.reshape(1, 1)

    eidx = edge_index.reshape(2 * ROWS, CH)

    deg = _make_sc_deg()(eidx)

    xt = x.T
    bd = pl.pallas_call(
        _tc_encbd_body,
        grid=(NG,),
        in_specs=[
            pl.BlockSpec((2 * NUM_TOKENS + 2, BN), lambda i: (0, i)),
            pl.BlockSpec((NC, BN, FW), lambda i: (0, i, 0)),
            pl.BlockSpec((2 * NUM_TOKENS + 2, FW), lambda i: (0, 0)),
            pl.BlockSpec((1, FW), lambda i: (0, 0)),
        ],
        out_specs=pl.BlockSpec((BN, FW), lambda i: (i, 0)),
        out_shape=jax.ShapeDtypeStruct((N_NODES, FW), f32),
    )(xt, deg, A, b_a)

    s_parts = _make_sc_edge()(eidx, bd)

    U = A @ W0
    biasU = b_a @ W0 + bias
    out = pl.pallas_call(
        _tc_final_body,
        grid=(NG,),
        in_specs=[
            pl.BlockSpec((2 * NUM_TOKENS + 2, BN), lambda i: (0, i)),
            pl.BlockSpec((BN, FW), lambda i: (i, 0)),
            pl.BlockSpec((NC, BN, FW), lambda i: (0, i, 0)),
            pl.BlockSpec((2 * NUM_TOKENS + 2, 128), lambda i: (0, 0)),
            pl.BlockSpec((FW, 128), lambda i: (0, 0)),
            pl.BlockSpec((1, 128), lambda i: (0, 0)),
            pl.BlockSpec((1, 64), lambda i: (0, 0)),
            pl.BlockSpec((1, 1), lambda i: (0, 0)),
        ],
        out_specs=pl.BlockSpec((BN, 1), lambda i: (i, 0)),
        out_shape=jax.ShapeDtypeStruct((N_NODES, 1), f32),
    )(xt, bd, s_parts, U, W1, biasU, lv, lb)

    return out


# speedup vs baseline: 1.0646x; 1.0646x over previous
import functools

import jax
import jax.numpy as jnp
from jax import lax
from jax.experimental import pallas as pl
from jax.experimental.pallas import tpu as pltpu
from jax.experimental.pallas import tpu_sc as plsc

N_NODES = 50000
N_EDGES = 800000
NUM_TOKENS = 157
FW = 16
NC, NS = 2, 16
NW = NC * NS
CH = 128
ROWS = N_EDGES // CH
ROWS_PW = ROWS // NW
ROWS_REM = ROWS - ROWS_PW * NW
NB = 24
NRND = 8
NBD = 15
NP = 50048
NPC = NP // NS
NPK = NP // 8
BN = 2048
NG = 25


def _sc_mesh():
    return plsc.VectorSubcoreMesh(core_axis_name="c", subcore_axis_name="s")


def _sc_params():
    return pltpu.CompilerParams(use_tc_tiling_on_sc=False)


def _zero_shared(zbuf, shared, s):
    @pl.loop(0, 128)
    def _(i):
        zbuf[i, :] = jnp.zeros((FW,), jnp.float32)

    @pl.loop(0, 24)
    def _(k):
        pltpu.sync_copy(zbuf, shared.at[pl.ds(s * NPC + k * 128, 128)])

    pltpu.sync_copy(zbuf.at[pl.ds(0, 104)],
                    shared.at[pl.ds(s * NPC + 24 * 128, 104)])


def _make_sc_deg():
    @functools.partial(
        pl.kernel,
        out_type=jax.ShapeDtypeStruct((NC, NP, FW), jnp.float32),
        mesh=_sc_mesh(),
        compiler_params=_sc_params(),
        scratch_types=[
            pltpu.VMEM((ROWS_PW, CH), jnp.int32),
            pltpu.VMEM((CH, FW), jnp.float32),
            pltpu.VMEM((128, FW), jnp.float32),
            pltpu.VMEM((CH,), jnp.int32),
            pltpu.VMEM_SHARED((NP, FW), jnp.float32),
            pltpu.SemaphoreType.DMA,
        ],
    )
    def sc_deg(eidx_hbm, deg_hbm, sidx, ones, zbuf, xsrc, shared, ssem):
        c = lax.axis_index("c")
        s = lax.axis_index("s")
        w = c * NS + s

        _zero_shared(zbuf, shared, s)

        @pl.loop(0, CH)
        def _(i):
            ones[i, :] = jnp.ones((FW,), jnp.float32)

        plsc.subcore_barrier()

        pltpu.sync_copy(eidx_hbm.at[pl.ds(w * ROWS_PW, ROWS_PW)], sidx)

        @pl.loop(0, ROWS_PW // NBD)
        def _(t):
            hs = [pltpu.async_copy(ones, shared.at[sidx.at[t * NBD + b]],
                                   ssem, add=True)
                  for b in range(NBD)]
            for h in hs:
                h.wait()

        @pl.when(w < ROWS_REM)
        def _():
            pltpu.sync_copy(eidx_hbm.at[NW * ROWS_PW + w], xsrc)
            pltpu.sync_copy(ones, shared.at[xsrc], add=True)

        plsc.subcore_barrier()
        pltpu.sync_copy(shared.at[pl.ds(s * NPC, NPC)],
                        deg_hbm.at[c, pl.ds(s * NPC, NPC)])

    return sc_deg


def _make_sc_edge():
    @functools.partial(
        pl.kernel,
        out_type=jax.ShapeDtypeStruct((NC, NP, FW), jnp.float32),
        mesh=_sc_mesh(),
        compiler_params=_sc_params(),
        scratch_types=[
            pltpu.VMEM((2, NB, CH), jnp.int32),
            pltpu.VMEM((2, NB, CH), jnp.int32),
            pltpu.VMEM((NB, CH, FW), jnp.float32),
            pltpu.VMEM((128, FW), jnp.float32),
            pltpu.VMEM((CH,), jnp.int32),
            pltpu.VMEM((CH,), jnp.int32),
            pltpu.VMEM_SHARED((NP, FW), jnp.float32),
            pltpu.SemaphoreType.DMA,
            pltpu.SemaphoreType.DMA,
            pltpu.SemaphoreType.DMA,
        ],
    )
    def sc_edge(eidx_hbm, bd_hbm, s_hbm,
                sidx, didx, rows, zbuf, xsrc, xdst, shared,
                gsem, ssem, isem):
        c = lax.axis_index("c")
        s = lax.axis_index("s")
        w = c * NS + s
        base = w * ROWS_PW

        def load_idx(t, slot):
            pltpu.async_copy(eidx_hbm.at[pl.ds(base + t * NB, NB)],
                             sidx.at[slot], isem)
            pltpu.async_copy(eidx_hbm.at[pl.ds(ROWS + base + t * NB, NB)],
                             didx.at[slot], isem)

        def wait_idx(slot):
            pltpu.make_async_copy(eidx_hbm.at[pl.ds(0, NB)],
                                  sidx.at[slot], isem).wait()
            pltpu.make_async_copy(eidx_hbm.at[pl.ds(0, NB)],
                                  didx.at[slot], isem).wait()

        load_idx(0, 0)
        _zero_shared(zbuf, shared, s)
        plsc.subcore_barrier()

        @pl.loop(0, NRND // 2)
        def _(tt):
            for slot in (0, 1):
                t = tt * 2 + slot
                wait_idx(slot)

                @pl.when(t < NRND - 1)
                def _():
                    load_idx(t + 1, 1 - slot)

                ghs = [pltpu.async_copy(bd_hbm.at[sidx.at[slot, b]],
                                        rows.at[b], gsem)
                       for b in range(NB)]
                shs = []
                for b in range(NB):
                    ghs[b].wait()
                    shs.append(pltpu.async_copy(
                        rows.at[b], shared.at[didx.at[slot, b]],
                        ssem, add=True))
                for h in shs:
                    h.wait()

        ntail = ROWS_PW - NRND * NB
        pltpu.sync_copy(eidx_hbm.at[pl.ds(base + NRND * NB, ntail)],
                        sidx.at[0, pl.ds(0, ntail)])
        pltpu.sync_copy(eidx_hbm.at[pl.ds(ROWS + base + NRND * NB, ntail)],
                        didx.at[0, pl.ds(0, ntail)])
        for b in range(ntail):
            pltpu.sync_copy(bd_hbm.at[sidx.at[0, b]], rows.at[b])
            pltpu.sync_copy(rows.at[b], shared.at[didx.at[0, b]],
                            add=True)

        @pl.when(w < ROWS_REM)
        def _():
            pltpu.sync_copy(eidx_hbm.at[NW * ROWS_PW + w], xsrc)
            pltpu.sync_copy(eidx_hbm.at[ROWS + NW * ROWS_PW + w], xdst)
            pltpu.sync_copy(bd_hbm.at[xsrc], rows.at[0])
            pltpu.sync_copy(rows.at[0], shared.at[xdst], add=True)

        plsc.subcore_barrier()
        pltpu.sync_copy(shared.at[pl.ds(s * NPC, NPC)],
                        s_hbm.at[c, pl.ds(s * NPC, NPC)])

    return sc_edge


def _tc_encode_body(xt_ref, a_ref, b_ref, o_ref):
    o_ref[...] = lax.dot_general(
        xt_ref[...], a_ref[...], (((0,), (0,)), ((), ())),
        preferred_element_type=jnp.float32) + b_ref[...]


def _tc_scale_body(a_ref, deg_ref, o_ref):
    d = deg_ref[0, :, 0:1] + deg_ref[1, :, 0:1]
    dis = jnp.where(d > 0.0, 1.0 / jnp.sqrt(jnp.maximum(d, 1.0)), 0.0)
    col = lax.broadcasted_iota(jnp.int32, (BN, FW), 1)
    o_ref[...] = jnp.where(col == 10, dis, a_ref[...] * dis)


def _tc_final_body(a_ref, bd_ref, s_ref, w0_ref, w1_ref, bias_ref,
                   lv_ref, lb_ref, o_ref):
    dis = bd_ref[:, 10:11]
    tx1 = -dis * (s_ref[0] + s_ref[1])
    g = (jnp.dot(a_ref[...], w0_ref[...], preferred_element_type=jnp.float32)
         + jnp.dot(tx1, w1_ref[...], preferred_element_type=jnp.float32)
         + bias_ref[...])
    z = jax.nn.sigmoid(g[:, 0:64])
    ht = jnp.tanh(g[:, 64:128])
    h = jax.nn.relu((1.0 - z) * ht)
    r = jnp.sum(h * lv_ref[...], axis=1, keepdims=True) + lb_ref[...]
    o_ref[...] = jax.nn.sigmoid(r)[:, 0]


def kernel(x, edge_index, enc_W, enc_b, Wxz0, Wxz1, bxz, Whz0, Whz1, bhz,
           Wxr0, Wxr1, bxr, Whr0, Whr1, bhr, Wxh0, Wxh1, bxh,
           Whh0, Whh1, bhh, lin_W, lin_b):
    f32 = jnp.float32
    A = jnp.zeros((2 * NUM_TOKENS + 2, FW), f32)
    A = A.at[0:NUM_TOKENS, 0:4].set(enc_W)
    A = A.at[NUM_TOKENS, 4].set(1.0)
    A = A.at[NUM_TOKENS + 1:2 * NUM_TOKENS + 1, 5:9].set(enc_W)
    A = A.at[2 * NUM_TOKENS + 1, 9].set(1.0)
    b_a = jnp.zeros((1, FW), f32)
    b_a = b_a.at[0, 0:4].set(enc_b).at[0, 5:9].set(enc_b)

    W0 = jnp.zeros((FW, 128), f32)
    W0 = W0.at[0:10, 0:64].set(Wxz0).at[0:10, 64:128].set(Wxh0)
    W1 = jnp.zeros((FW, 128), f32)
    W1 = W1.at[0:10, 0:64].set(Wxz1).at[0:10, 64:128].set(Wxh1)
    bias = jnp.concatenate([bxz + bhz, bxh + bhh]).reshape(1, 128)
    lv = lin_W.reshape(1, 64)
    lb = lin_b.reshape(1, 1)

    eidx = edge_index.reshape(2 * ROWS, CH)

    deg = _make_sc_deg()(eidx)

    xt = x.T
    a_pad = pl.pallas_call(
        _tc_encode_body,
        grid=(NG,),
        in_specs=[
            pl.BlockSpec((2 * NUM_TOKENS + 2, BN), lambda i: (0, i)),
            pl.BlockSpec((2 * NUM_TOKENS + 2, FW), lambda i: (0, 0)),
            pl.BlockSpec((1, FW), lambda i: (0, 0)),
        ],
        out_specs=pl.BlockSpec((BN, FW), lambda i: (i, 0)),
        out_shape=jax.ShapeDtypeStruct((N_NODES, FW), f32),
    )(xt, A, b_a)

    bd = pl.pallas_call(
        _tc_scale_body,
        grid=(NG,),
        in_specs=[
            pl.BlockSpec((BN, FW), lambda i: (i, 0)),
            pl.BlockSpec((NC, BN, FW), lambda i: (0, i, 0)),
        ],
        out_specs=pl.BlockSpec((BN, FW), lambda i: (i, 0)),
        out_shape=jax.ShapeDtypeStruct((N_NODES, FW), f32),
    )(a_pad, deg)

    s_parts = _make_sc_edge()(eidx, bd)

    out = pl.pallas_call(
        _tc_final_body,
        grid=(NG,),
        in_specs=[
            pl.BlockSpec((BN, FW), lambda i: (i, 0)),
            pl.BlockSpec((BN, FW), lambda i: (i, 0)),
            pl.BlockSpec((NC, BN, FW), lambda i: (0, i, 0)),
            pl.BlockSpec((FW, 128), lambda i: (0, 0)),
            pl.BlockSpec((FW, 128), lambda i: (0, 0)),
            pl.BlockSpec((1, 128), lambda i: (0, 0)),
            pl.BlockSpec((1, 64), lambda i: (0, 0)),
            pl.BlockSpec((1, 1), lambda i: (0, 0)),
        ],
        out_specs=pl.BlockSpec((BN,), lambda i: (i,)),
        out_shape=jax.ShapeDtypeStruct((N_NODES,), f32),
    )(a_pad, bd, s_parts, W0, W1, bias, lv, lb)

    return out.reshape(N_NODES, 1)


# speedup vs baseline: 1.0855x; 1.0197x over previous
import functools

import jax
import jax.numpy as jnp
from jax import lax
from jax.experimental import pallas as pl
from jax.experimental.pallas import tpu as pltpu
from jax.experimental.pallas import tpu_sc as plsc

N_NODES = 50000
N_EDGES = 800000
NUM_TOKENS = 157
FW = 16
NC, NS = 2, 16
NW = NC * NS
CH = 128
ROWS = N_EDGES // CH
ROWS_PW = ROWS // NW
ROWS_REM = ROWS - ROWS_PW * NW
NB = 24
NRND = 8
NBD = 15
NP = 50048
NPC = NP // NS
NPK = NP // 8
BN = 2048
NG = 25
BNE = 4096


def _sc_mesh():
    return plsc.VectorSubcoreMesh(core_axis_name="c", subcore_axis_name="s")


def _sc_params():
    return pltpu.CompilerParams(use_tc_tiling_on_sc=False)


def _zero_shared(zbuf, shared, s):
    @pl.loop(0, 128)
    def _(i):
        zbuf[i, :] = jnp.zeros((FW,), jnp.float32)

    @pl.loop(0, 24)
    def _(k):
        pltpu.sync_copy(zbuf, shared.at[pl.ds(s * NPC + k * 128, 128)])

    pltpu.sync_copy(zbuf.at[pl.ds(0, 104)],
                    shared.at[pl.ds(s * NPC + 24 * 128, 104)])


def _make_sc_deg():
    @functools.partial(
        pl.kernel,
        out_type=jax.ShapeDtypeStruct((NC, NP, FW), jnp.float32),
        mesh=_sc_mesh(),
        compiler_params=_sc_params(),
        scratch_types=[
            pltpu.VMEM((ROWS_PW, CH), jnp.int32),
            pltpu.VMEM((CH, FW), jnp.float32),
            pltpu.VMEM((128, FW), jnp.float32),
            pltpu.VMEM((CH,), jnp.int32),
            pltpu.VMEM_SHARED((NP, FW), jnp.float32),
            pltpu.SemaphoreType.DMA,
        ],
    )
    def sc_deg(eidx_hbm, deg_hbm, sidx, ones, zbuf, xsrc, shared, ssem):
        c = lax.axis_index("c")
        s = lax.axis_index("s")
        w = c * NS + s

        _zero_shared(zbuf, shared, s)

        @pl.loop(0, CH)
        def _(i):
            ones[i, :] = jnp.ones((FW,), jnp.float32)

        plsc.subcore_barrier()

        pltpu.sync_copy(eidx_hbm.at[pl.ds(w * ROWS_PW, ROWS_PW)], sidx)

        @pl.loop(0, ROWS_PW // NBD)
        def _(t):
            hs = [pltpu.async_copy(ones, shared.at[sidx.at[t * NBD + b]],
                                   ssem, add=True)
                  for b in range(NBD)]
            for h in hs:
                h.wait()

        @pl.when(w < ROWS_REM)
        def _():
            pltpu.sync_copy(eidx_hbm.at[NW * ROWS_PW + w], xsrc)
            pltpu.sync_copy(ones, shared.at[xsrc], add=True)

        plsc.subcore_barrier()
        pltpu.sync_copy(shared.at[pl.ds(s * NPC, NPC)],
                        deg_hbm.at[c, pl.ds(s * NPC, NPC)])

    return sc_deg


def _make_sc_edge():
    @functools.partial(
        pl.kernel,
        out_type=jax.ShapeDtypeStruct((NC, NP, FW), jnp.float32),
        mesh=_sc_mesh(),
        compiler_params=_sc_params(),
        scratch_types=[
            pltpu.VMEM((2, NB, CH), jnp.int32),
            pltpu.VMEM((2, NB, CH), jnp.int32),
            pltpu.VMEM((NB, CH, FW), jnp.float32),
            pltpu.VMEM((128, FW), jnp.float32),
            pltpu.VMEM((CH,), jnp.int32),
            pltpu.VMEM((CH,), jnp.int32),
            pltpu.VMEM_SHARED((NP, FW), jnp.float32),
            pltpu.SemaphoreType.DMA,
            pltpu.SemaphoreType.DMA,
            pltpu.SemaphoreType.DMA,
        ],
    )
    def sc_edge(eidx_hbm, bd_hbm, s_hbm,
                sidx, didx, rows, zbuf, xsrc, xdst, shared,
                gsem, ssem, isem):
        c = lax.axis_index("c")
        s = lax.axis_index("s")
        w = c * NS + s
        base = w * ROWS_PW

        def load_idx(t, slot):
            pltpu.async_copy(eidx_hbm.at[pl.ds(base + t * NB, NB)],
                             sidx.at[slot], isem)
            pltpu.async_copy(eidx_hbm.at[pl.ds(ROWS + base + t * NB, NB)],
                             didx.at[slot], isem)

        def wait_idx(slot):
            pltpu.make_async_copy(eidx_hbm.at[pl.ds(0, NB)],
                                  sidx.at[slot], isem).wait()
            pltpu.make_async_copy(eidx_hbm.at[pl.ds(0, NB)],
                                  didx.at[slot], isem).wait()

        load_idx(0, 0)
        _zero_shared(zbuf, shared, s)
        plsc.subcore_barrier()

        @pl.loop(0, NRND // 2)
        def _(tt):
            for slot in (0, 1):
                t = tt * 2 + slot
                wait_idx(slot)

                @pl.when(t < NRND - 1)
                def _():
                    load_idx(t + 1, 1 - slot)

                ghs = [pltpu.async_copy(bd_hbm.at[sidx.at[slot, b]],
                                        rows.at[b], gsem)
                       for b in range(NB)]
                shs = []
                for b in range(NB):
                    ghs[b].wait()
                    shs.append(pltpu.async_copy(
                        rows.at[b], shared.at[didx.at[slot, b]],
                        ssem, add=True))
                for h in shs:
                    h.wait()

        ntail = ROWS_PW - NRND * NB
        pltpu.sync_copy(eidx_hbm.at[pl.ds(base + NRND * NB, ntail)],
                        sidx.at[0, pl.ds(0, ntail)])
        pltpu.sync_copy(eidx_hbm.at[pl.ds(ROWS + base + NRND * NB, ntail)],
                        didx.at[0, pl.ds(0, ntail)])
        for b in range(ntail):
            pltpu.sync_copy(bd_hbm.at[sidx.at[0, b]], rows.at[b])
            pltpu.sync_copy(rows.at[b], shared.at[didx.at[0, b]],
                            add=True)

        @pl.when(w < ROWS_REM)
        def _():
            pltpu.sync_copy(eidx_hbm.at[NW * ROWS_PW + w], xsrc)
            pltpu.sync_copy(eidx_hbm.at[ROWS + NW * ROWS_PW + w], xdst)
            pltpu.sync_copy(bd_hbm.at[xsrc], rows.at[0])
            pltpu.sync_copy(rows.at[0], shared.at[xdst], add=True)

        plsc.subcore_barrier()
        pltpu.sync_copy(shared.at[pl.ds(s * NPC, NPC)],
                        s_hbm.at[c, pl.ds(s * NPC, NPC)])

    return sc_edge


def _tc_encode_body(xt_ref, a_ref, b_ref, o_ref):
    o_ref[...] = lax.dot_general(
        xt_ref[...], a_ref[...], (((0,), (0,)), ((), ())),
        preferred_element_type=jnp.float32) + b_ref[...]


def _tc_scale_body(a_ref, deg_ref, o_ref):
    d = deg_ref[0, :, 0:1] + deg_ref[1, :, 0:1]
    dis = jnp.where(d > 0.0, 1.0 / jnp.sqrt(jnp.maximum(d, 1.0)), 0.0)
    col = lax.broadcasted_iota(jnp.int32, (BN, FW), 1)
    o_ref[...] = jnp.where(col == 10, dis, a_ref[...] * dis)


def _tc_final_body(a_ref, bd_ref, s_ref, w0_ref, w1_ref, bias_ref,
                   lv_ref, lb_ref, o_ref):
    dis = bd_ref[:, 10:11]
    tx1 = -dis * (s_ref[0] + s_ref[1])
    g = (jnp.dot(a_ref[...], w0_ref[...], preferred_element_type=jnp.float32)
         + jnp.dot(tx1, w1_ref[...], preferred_element_type=jnp.float32)
         + bias_ref[...])
    z = jax.nn.sigmoid(g[:, 0:64])
    ht = jnp.tanh(g[:, 64:128])
    h = jax.nn.relu((1.0 - z) * ht)
    r = jnp.sum(h * lv_ref[...], axis=1, keepdims=True) + lb_ref[...]
    o_ref[...] = jax.nn.sigmoid(r)[:, 0]


def kernel(x, edge_index, enc_W, enc_b, Wxz0, Wxz1, bxz, Whz0, Whz1, bhz,
           Wxr0, Wxr1, bxr, Whr0, Whr1, bhr, Wxh0, Wxh1, bxh,
           Whh0, Whh1, bhh, lin_W, lin_b):
    f32 = jnp.float32
    A = jnp.zeros((2 * NUM_TOKENS + 2, FW), f32)
    A = A.at[0:NUM_TOKENS, 0:4].set(enc_W)
    A = A.at[NUM_TOKENS, 4].set(1.0)
    A = A.at[NUM_TOKENS + 1:2 * NUM_TOKENS + 1, 5:9].set(enc_W)
    A = A.at[2 * NUM_TOKENS + 1, 9].set(1.0)
    b_a = jnp.zeros((1, FW), f32)
    b_a = b_a.at[0, 0:4].set(enc_b).at[0, 5:9].set(enc_b)

    W0 = jnp.zeros((FW, 128), f32)
    W0 = W0.at[0:10, 0:64].set(Wxz0).at[0:10, 64:128].set(Wxh0)
    W1 = jnp.zeros((FW, 128), f32)
    W1 = W1.at[0:10, 0:64].set(Wxz1).at[0:10, 64:128].set(Wxh1)
    bias = jnp.concatenate([bxz + bhz, bxh + bhh]).reshape(1, 128)
    lv = lin_W.reshape(1, 64)
    lb = lin_b.reshape(1, 1)

    eidx = edge_index.reshape(2 * ROWS, CH)

    deg = _make_sc_deg()(eidx)

    xt = x.T
    a_pad = pl.pallas_call(
        _tc_encode_body,
        grid=(13,),
        in_specs=[
            pl.BlockSpec((2 * NUM_TOKENS + 2, BNE), lambda i: (0, i)),
            pl.BlockSpec((2 * NUM_TOKENS + 2, FW), lambda i: (0, 0)),
            pl.BlockSpec((1, FW), lambda i: (0, 0)),
        ],
        out_specs=pl.BlockSpec((BNE, FW), lambda i: (i, 0)),
        out_shape=jax.ShapeDtypeStruct((N_NODES, FW), f32),
    )(xt, A, b_a)

    bd = pl.pallas_call(
        _tc_scale_body,
        grid=(NG,),
        in_specs=[
            pl.BlockSpec((BN, FW), lambda i: (i, 0)),
            pl.BlockSpec((NC, BN, FW), lambda i: (0, i, 0)),
        ],
        out_specs=pl.BlockSpec((BN, FW), lambda i: (i, 0)),
        out_shape=jax.ShapeDtypeStruct((N_NODES, FW), f32),
    )(a_pad, deg)

    s_parts = _make_sc_edge()(eidx, bd)

    out = pl.pallas_call(
        _tc_final_body,
        grid=(NG,),
        in_specs=[
            pl.BlockSpec((BN, FW), lambda i: (i, 0)),
            pl.BlockSpec((BN, FW), lambda i: (i, 0)),
            pl.BlockSpec((NC, BN, FW), lambda i: (0, i, 0)),
            pl.BlockSpec((FW, 128), lambda i: (0, 0)),
            pl.BlockSpec((FW, 128), lambda i: (0, 0)),
            pl.BlockSpec((1, 128), lambda i: (0, 0)),
            pl.BlockSpec((1, 64), lambda i: (0, 0)),
            pl.BlockSpec((1, 1), lambda i: (0, 0)),
        ],
        out_specs=pl.BlockSpec((BN,), lambda i: (i,)),
        out_shape=jax.ShapeDtypeStruct((N_NODES,), f32),
    )(a_pad, bd, s_parts, W0, W1, bias, lv, lb)

    return out.reshape(N_NODES, 1)


# speedup vs baseline: 1.1155x; 1.0276x over previous
import functools

import jax
import jax.numpy as jnp
from jax import lax
from jax.experimental import pallas as pl
from jax.experimental.pallas import tpu as pltpu
from jax.experimental.pallas import tpu_sc as plsc

N_NODES = 50000
N_EDGES = 800000
NUM_TOKENS = 157
FW = 16
NC, NS = 2, 16
NW = NC * NS
CH = 128
ROWS = N_EDGES // CH
ROWS_PW = ROWS // NW
ROWS_REM = ROWS - ROWS_PW * NW
NB = 24
NRND = 8
NBD = 15
NP = 50048
NPC = NP // NS
NPK = NP // 8
BN = 4096
NG = 13
BNE = 4096


def _sc_mesh():
    return plsc.VectorSubcoreMesh(core_axis_name="c", subcore_axis_name="s")


def _sc_params():
    return pltpu.CompilerParams(use_tc_tiling_on_sc=False)


def _zero_shared(zbuf, shared, s):
    @pl.loop(0, 128)
    def _(i):
        zbuf[i, :] = jnp.zeros((FW,), jnp.float32)

    @pl.loop(0, 24)
    def _(k):
        pltpu.sync_copy(zbuf, shared.at[pl.ds(s * NPC + k * 128, 128)])

    pltpu.sync_copy(zbuf.at[pl.ds(0, 104)],
                    shared.at[pl.ds(s * NPC + 24 * 128, 104)])


def _make_sc_deg():
    @functools.partial(
        pl.kernel,
        out_type=jax.ShapeDtypeStruct((NC, NP, FW), jnp.float32),
        mesh=_sc_mesh(),
        compiler_params=_sc_params(),
        scratch_types=[
            pltpu.VMEM((ROWS_PW, CH), jnp.int32),
            pltpu.VMEM((CH, FW), jnp.float32),
            pltpu.VMEM((128, FW), jnp.float32),
            pltpu.VMEM((CH,), jnp.int32),
            pltpu.VMEM_SHARED((NP, FW), jnp.float32),
            pltpu.SemaphoreType.DMA,
        ],
    )
    def sc_deg(eidx_hbm, deg_hbm, sidx, ones, zbuf, xsrc, shared, ssem):
        c = lax.axis_index("c")
        s = lax.axis_index("s")
        w = c * NS + s

        _zero_shared(zbuf, shared, s)

        @pl.loop(0, CH)
        def _(i):
            ones[i, :] = jnp.ones((FW,), jnp.float32)

        plsc.subcore_barrier()

        pltpu.sync_copy(eidx_hbm.at[pl.ds(w * ROWS_PW, ROWS_PW)], sidx)

        @pl.loop(0, ROWS_PW // NBD)
        def _(t):
            hs = [pltpu.async_copy(ones, shared.at[sidx.at[t * NBD + b]],
                                   ssem, add=True)
                  for b in range(NBD)]
            for h in hs:
                h.wait()

        @pl.when(w < ROWS_REM)
        def _():
            pltpu.sync_copy(eidx_hbm.at[NW * ROWS_PW + w], xsrc)
            pltpu.sync_copy(ones, shared.at[xsrc], add=True)

        plsc.subcore_barrier()
        pltpu.sync_copy(shared.at[pl.ds(s * NPC, NPC)],
                        deg_hbm.at[c, pl.ds(s * NPC, NPC)])

    return sc_deg


def _make_sc_edge():
    @functools.partial(
        pl.kernel,
        out_type=jax.ShapeDtypeStruct((NC, NP, FW), jnp.float32),
        mesh=_sc_mesh(),
        compiler_params=_sc_params(),
        scratch_types=[
            pltpu.VMEM((2, NB, CH), jnp.int32),
            pltpu.VMEM((2, NB, CH), jnp.int32),
            pltpu.VMEM((NB, CH, FW), jnp.float32),
            pltpu.VMEM((128, FW), jnp.float32),
            pltpu.VMEM((CH,), jnp.int32),
            pltpu.VMEM((CH,), jnp.int32),
            pltpu.VMEM_SHARED((NP, FW), jnp.float32),
            pltpu.SemaphoreType.DMA,
            pltpu.SemaphoreType.DMA,
            pltpu.SemaphoreType.DMA,
        ],
    )
    def sc_edge(eidx_hbm, bd_hbm, s_hbm,
                sidx, didx, rows, zbuf, xsrc, xdst, shared,
                gsem, ssem, isem):
        c = lax.axis_index("c")
        s = lax.axis_index("s")
        w = c * NS + s
        base = w * ROWS_PW

        def load_idx(t, slot):
            pltpu.async_copy(eidx_hbm.at[pl.ds(base + t * NB, NB)],
                             sidx.at[slot], isem)
            pltpu.async_copy(eidx_hbm.at[pl.ds(ROWS + base + t * NB, NB)],
                             didx.at[slot], isem)

        def wait_idx(slot):
            pltpu.make_async_copy(eidx_hbm.at[pl.ds(0, NB)],
                                  sidx.at[slot], isem).wait()
            pltpu.make_async_copy(eidx_hbm.at[pl.ds(0, NB)],
                                  didx.at[slot], isem).wait()

        load_idx(0, 0)
        _zero_shared(zbuf, shared, s)
        plsc.subcore_barrier()

        @pl.loop(0, NRND // 2)
        def _(tt):
            for slot in (0, 1):
                t = tt * 2 + slot
                wait_idx(slot)

                @pl.when(t < NRND - 1)
                def _():
                    load_idx(t + 1, 1 - slot)

                ghs = [pltpu.async_copy(bd_hbm.at[sidx.at[slot, b]],
                                        rows.at[b], gsem)
                       for b in range(NB)]
                shs = []
                for b in range(NB):
                    ghs[b].wait()
                    shs.append(pltpu.async_copy(
                        rows.at[b], shared.at[didx.at[slot, b]],
                        ssem, add=True))
                for h in shs:
                    h.wait()

        ntail = ROWS_PW - NRND * NB
        pltpu.sync_copy(eidx_hbm.at[pl.ds(base + NRND * NB, ntail)],
                        sidx.at[0, pl.ds(0, ntail)])
        pltpu.sync_copy(eidx_hbm.at[pl.ds(ROWS + base + NRND * NB, ntail)],
                        didx.at[0, pl.ds(0, ntail)])
        for b in range(ntail):
            pltpu.sync_copy(bd_hbm.at[sidx.at[0, b]], rows.at[b])
            pltpu.sync_copy(rows.at[b], shared.at[didx.at[0, b]],
                            add=True)

        @pl.when(w < ROWS_REM)
        def _():
            pltpu.sync_copy(eidx_hbm.at[NW * ROWS_PW + w], xsrc)
            pltpu.sync_copy(eidx_hbm.at[ROWS + NW * ROWS_PW + w], xdst)
            pltpu.sync_copy(bd_hbm.at[xsrc], rows.at[0])
            pltpu.sync_copy(rows.at[0], shared.at[xdst], add=True)

        plsc.subcore_barrier()
        pltpu.sync_copy(shared.at[pl.ds(s * NPC, NPC)],
                        s_hbm.at[c, pl.ds(s * NPC, NPC)])

    return sc_edge


def _tc_encode_body(xt_ref, a_ref, b_ref, o_ref):
    o_ref[...] = lax.dot_general(
        xt_ref[...], a_ref[...], (((0,), (0,)), ((), ())),
        preferred_element_type=jnp.float32) + b_ref[...]


def _tc_scale_body(a_ref, deg_ref, o_ref):
    d = deg_ref[0, :, 0:1] + deg_ref[1, :, 0:1]
    dis = jnp.where(d > 0.0, 1.0 / jnp.sqrt(jnp.maximum(d, 1.0)), 0.0)
    col = lax.broadcasted_iota(jnp.int32, (BN, FW), 1)
    o_ref[...] = jnp.where(col == 10, dis, a_ref[...] * dis)


def _tc_final_body(a_ref, bd_ref, s_ref, w0_ref, w1_ref, bias_ref,
                   lv_ref, lb_ref, o_ref):
    dis = bd_ref[:, 10:11]
    tx1 = -dis * (s_ref[0] + s_ref[1])
    g = (jnp.dot(a_ref[...], w0_ref[...], preferred_element_type=jnp.float32)
         + jnp.dot(tx1, w1_ref[...], preferred_element_type=jnp.float32)
         + bias_ref[...])
    z = jax.nn.sigmoid(g[:, 0:64])
    ht = jnp.tanh(g[:, 64:128])
    h = jax.nn.relu((1.0 - z) * ht)
    r = jnp.sum(h * lv_ref[...], axis=1, keepdims=True) + lb_ref[...]
    o_ref[...] = jax.nn.sigmoid(r)[:, 0]


def kernel(x, edge_index, enc_W, enc_b, Wxz0, Wxz1, bxz, Whz0, Whz1, bhz,
           Wxr0, Wxr1, bxr, Whr0, Whr1, bhr, Wxh0, Wxh1, bxh,
           Whh0, Whh1, bhh, lin_W, lin_b):
    f32 = jnp.float32
    A = jnp.zeros((2 * NUM_TOKENS + 2, FW), f32)
    A = A.at[0:NUM_TOKENS, 0:4].set(enc_W)
    A = A.at[NUM_TOKENS, 4].set(1.0)
    A = A.at[NUM_TOKENS + 1:2 * NUM_TOKENS + 1, 5:9].set(enc_W)
    A = A.at[2 * NUM_TOKENS + 1, 9].set(1.0)
    b_a = jnp.zeros((1, FW), f32)
    b_a = b_a.at[0, 0:4].set(enc_b).at[0, 5:9].set(enc_b)

    W0 = jnp.zeros((FW, 128), f32)
    W0 = W0.at[0:10, 0:64].set(Wxz0).at[0:10, 64:128].set(Wxh0)
    W1 = jnp.zeros((FW, 128), f32)
    W1 = W1.at[0:10, 0:64].set(Wxz1).at[0:10, 64:128].set(Wxh1)
    bias = jnp.concatenate([bxz + bhz, bxh + bhh]).reshape(1, 128)
    lv = lin_W.reshape(1, 64)
    lb = lin_b.reshape(1, 1)

    eidx = edge_index.reshape(2 * ROWS, CH)

    deg = _make_sc_deg()(eidx)

    xt = x.T
    a_pad = pl.pallas_call(
        _tc_encode_body,
        grid=(13,),
        in_specs=[
            pl.BlockSpec((2 * NUM_TOKENS + 2, BNE), lambda i: (0, i)),
            pl.BlockSpec((2 * NUM_TOKENS + 2, FW), lambda i: (0, 0)),
            pl.BlockSpec((1, FW), lambda i: (0, 0)),
        ],
        out_specs=pl.BlockSpec((BNE, FW), lambda i: (i, 0)),
        out_shape=jax.ShapeDtypeStruct((N_NODES, FW), f32),
    )(xt, A, b_a)

    bd = pl.pallas_call(
        _tc_scale_body,
        grid=(NG,),
        in_specs=[
            pl.BlockSpec((BN, FW), lambda i: (i, 0)),
            pl.BlockSpec((NC, BN, FW), lambda i: (0, i, 0)),
        ],
        out_specs=pl.BlockSpec((BN, FW), lambda i: (i, 0)),
        out_shape=jax.ShapeDtypeStruct((N_NODES, FW), f32),
    )(a_pad, deg)

    s_parts = _make_sc_edge()(eidx, bd)

    out = pl.pallas_call(
        _tc_final_body,
        grid=(NG,),
        in_specs=[
            pl.BlockSpec((BN, FW), lambda i: (i, 0)),
            pl.BlockSpec((BN, FW), lambda i: (i, 0)),
            pl.BlockSpec((NC, BN, FW), lambda i: (0, i, 0)),
            pl.BlockSpec((FW, 128), lambda i: (0, 0)),
            pl.BlockSpec((FW, 128), lambda i: (0, 0)),
            pl.BlockSpec((1, 128), lambda i: (0, 0)),
            pl.BlockSpec((1, 64), lambda i: (0, 0)),
            pl.BlockSpec((1, 1), lambda i: (0, 0)),
        ],
        out_specs=pl.BlockSpec((BN,), lambda i: (i,)),
        out_shape=jax.ShapeDtypeStruct((N_NODES,), f32),
    )(a_pad, bd, s_parts, W0, W1, bias, lv, lb)

    return out.reshape(N_NODES, 1)
